# baseline (speedup 1.0000x reference)
import functools

import jax
import jax.numpy as jnp
from jax import lax
from jax.experimental import pallas as pl
from jax.experimental.pallas import tpu as pltpu
from jax.experimental.pallas import tpu_sc as plsc

N_NODES = 10000
N_EDGES = 320000
D = 128

NC, NS = 2, 16
CHUNK = 64
CPT = 160
E_PAD = NC * NS * CPT * CHUNK
N_CHUNKS = E_PAD // CHUNK
ROWS_PER_TILE = 632
ACC_ROWS = NS * ROWS_PER_TILE
X_TAIL = N_NODES - 15 * ROWS_PER_TILE
ZROWS = 8
NBUF = 4
IDX_STAGE = 32


def _sc_body(s2, r2, x_hbm, out_hbm, acc, sidx, ridx, rows, zbuf, sem, sem_s,
             sem_i):
    c = lax.axis_index("c")
    w = lax.axis_index("s")
    gw = c * NS + w
    r0 = gw * CPT

    def stage_start(h):
        b = lax.rem(h, 2)
        pltpu.async_copy(s2.at[pl.ds(r0 + h * IDX_STAGE, IDX_STAGE)],
                         sidx.at[b], sem_i)
        pltpu.async_copy(r2.at[pl.ds(r0 + h * IDX_STAGE, IDX_STAGE)],
                         ridx.at[b], sem_i)

    def stage_wait(h):
        b = lax.rem(h, 2)
        pltpu.make_async_copy(s2.at[pl.ds(r0 + h * IDX_STAGE, IDX_STAGE)],
                              sidx.at[b], sem_i).wait()
        pltpu.make_async_copy(r2.at[pl.ds(r0 + h * IDX_STAGE, IDX_STAGE)],
                              ridx.at[b], sem_i).wait()

    def start_gather(j, s):
        pltpu.async_copy(
            x_hbm.at[sidx.at[lax.rem(j // IDX_STAGE, 2), lax.rem(j, IDX_STAGE)]],
            rows.at[s], sem)

    def wait_gather(j, s):
        pltpu.make_async_copy(
            x_hbm.at[sidx.at[lax.rem(j // IDX_STAGE, 2), lax.rem(j, IDX_STAGE)]],
            rows.at[s], sem).wait()

    def start_scatter(j, s):
        pltpu.async_copy(
            rows.at[s],
            acc.at[ridx.at[lax.rem(j // IDX_STAGE, 2), lax.rem(j, IDX_STAGE)]],
            sem_s, add=True)

    def wait_scatter(j, s):
        pltpu.make_async_copy(
            rows.at[s],
            acc.at[ridx.at[lax.rem(j // IDX_STAGE, 2), lax.rem(j, IDX_STAGE)]],
            sem_s).wait()

    stage_start(0)

    for i in range(ZROWS):
        for j in range(D // 16):
            zbuf[i, pl.ds(j * 16, 16)] = jnp.zeros((16,), jnp.float32)

    @pl.when((c == 0) & (w < NS - 1))
    def _():
        pltpu.sync_copy(x_hbm.at[pl.ds(w * ROWS_PER_TILE, ROWS_PER_TILE)],
                        acc.at[pl.ds(w * ROWS_PER_TILE, ROWS_PER_TILE)])

    @pl.when((c == 0) & (w == NS - 1))
    def _():
        pltpu.sync_copy(x_hbm.at[pl.ds(w * ROWS_PER_TILE, X_TAIL)],
                        acc.at[pl.ds(w * ROWS_PER_TILE, X_TAIL)])
        for k in range((ROWS_PER_TILE - X_TAIL) // ZROWS):
            pltpu.sync_copy(
                zbuf, acc.at[pl.ds(w * ROWS_PER_TILE + X_TAIL + k * ZROWS, ZROWS)])

    @pl.when(c != 0)
    def _():
        for k in range(ROWS_PER_TILE // ZROWS):
            pltpu.sync_copy(
                zbuf, acc.at[pl.ds(w * ROWS_PER_TILE + k * ZROWS, ZROWS)])

    stage_wait(0)
    for p in range(NBUF - 1):
        start_gather(p, p)

    plsc.subcore_barrier()

    def step(g, carry):
        j0 = g * NBUF
        for u in range(NBUF):
            j = j0 + u
            wait_gather(j, u)
            start_scatter(j, u)
            if u == 0:
                @pl.when(j >= 1)
                def _():
                    wait_scatter(j - 1, NBUF - 1)

                @pl.when((lax.rem(g, IDX_STAGE // NBUF) == 1)
                         & (j + IDX_STAGE - NBUF < CPT))
                def _():
                    stage_start(j // IDX_STAGE + 1)
            else:
                wait_scatter(j - 1, u - 1)

            jn = j + NBUF - 1

            @pl.when(jn < CPT)
            def _():
                if u == 1:
                    @pl.when(lax.rem(g, IDX_STAGE // NBUF)
                             == IDX_STAGE // NBUF - 1)
                    def _():
                        stage_wait(jn // IDX_STAGE)

                start_gather(jn, (u + NBUF - 1) % NBUF)
        return carry

    lax.fori_loop(0, CPT // NBUF, step, 0)
    wait_scatter(CPT - 1, (CPT - 1) % NBUF)

    plsc.subcore_barrier()

    pltpu.sync_copy(acc.at[pl.ds(w * ROWS_PER_TILE, ROWS_PER_TILE)],
                    out_hbm.at[c, pl.ds(w * ROWS_PER_TILE, ROWS_PER_TILE)])


@functools.partial(
    pl.kernel,
    out_type=jax.ShapeDtypeStruct((NC, ACC_ROWS, D), jnp.float32),
    mesh=plsc.VectorSubcoreMesh(core_axis_name="c", subcore_axis_name="s"),
    scratch_types=[
        pltpu.VMEM_SHARED((ACC_ROWS, D), jnp.float32),
        pltpu.VMEM((2, IDX_STAGE, CHUNK), jnp.int32),
        pltpu.VMEM((2, IDX_STAGE, CHUNK), jnp.int32),
        pltpu.VMEM((NBUF, CHUNK, D), jnp.float32),
        pltpu.VMEM((ZROWS, D), jnp.float32),
        pltpu.SemaphoreType.DMA,
        pltpu.SemaphoreType.DMA,
        pltpu.SemaphoreType.DMA,
    ],
)
def _mpnn_sc(s2, r2, x_hbm, out_hbm, acc, sidx, ridx, rows, zbuf, sem, sem_s,
             sem_i):
    _sc_body(s2, r2, x_hbm, out_hbm, acc, sidx, ridx, rows, zbuf, sem, sem_s,
             sem_i)


def _combine_body(p_ref, o_ref):
    o_ref[...] = p_ref[0] + p_ref[1]


_combine = pl.pallas_call(
    _combine_body,
    grid=(10,),
    in_specs=[pl.BlockSpec((NC, N_NODES // 10, D), lambda i: (0, i, 0))],
    out_specs=pl.BlockSpec((N_NODES // 10, D), lambda i: (i, 0)),
    out_shape=jax.ShapeDtypeStruct((N_NODES, D), jnp.float32),
)


def kernel(V, E, X):
    pad = E_PAD - N_EDGES
    pad_send = jnp.arange(pad, dtype=jnp.int32) % N_NODES
    pad_recv = N_NODES + jnp.arange(pad, dtype=jnp.int32) % (ACC_ROWS - N_NODES)
    senders = jnp.concatenate([E[0], pad_send]).reshape(N_CHUNKS, CHUNK)
    receivers = jnp.concatenate([E[1], pad_recv]).reshape(N_CHUNKS, CHUNK)
    partials = _mpnn_sc(senders, receivers, X)
    return _combine(partials)


# speedup vs baseline: 1.0002x; 1.0002x over previous
import functools

import jax
import jax.numpy as jnp
from jax import lax
from jax.experimental import pallas as pl
from jax.experimental.pallas import tpu as pltpu
from jax.experimental.pallas import tpu_sc as plsc

N_NODES = 10000
N_EDGES = 320000
D = 128

NC, NS = 2, 16
CHUNK = 64
CPT = 160
E_PAD = NC * NS * CPT * CHUNK
N_CHUNKS = E_PAD // CHUNK
ROWS_PER_TILE = 632
ACC_ROWS = NS * ROWS_PER_TILE
X_TAIL = N_NODES - 15 * ROWS_PER_TILE
ZROWS = 8
NBUF = 4
IDX_STAGE = 32


def _sc_body(s2, r2, x_hbm, out_hbm, acc, sidx, ridx, rows, zbuf, sem, sem_s,
             sem_i):
    c = lax.axis_index("c")
    w = lax.axis_index("s")
    gw = c * NS + w
    r0 = gw * CPT

    def stage_start(h):
        b = lax.rem(h, 2)
        pltpu.async_copy(s2.at[pl.ds(r0 + h * IDX_STAGE, IDX_STAGE)],
                         sidx.at[b], sem_i)
        pltpu.async_copy(r2.at[pl.ds(r0 + h * IDX_STAGE, IDX_STAGE)],
                         ridx.at[b], sem_i)

    def stage_wait(h):
        b = lax.rem(h, 2)
        pltpu.make_async_copy(s2.at[pl.ds(r0 + h * IDX_STAGE, IDX_STAGE)],
                              sidx.at[b], sem_i).wait()
        pltpu.make_async_copy(r2.at[pl.ds(r0 + h * IDX_STAGE, IDX_STAGE)],
                              ridx.at[b], sem_i).wait()

    def start_gather(j, s):
        pltpu.async_copy(
            x_hbm.at[sidx.at[lax.rem(j // IDX_STAGE, 2), lax.rem(j, IDX_STAGE)]],
            rows.at[s], sem)

    def wait_gather(j, s):
        pltpu.make_async_copy(
            x_hbm.at[sidx.at[lax.rem(j // IDX_STAGE, 2), lax.rem(j, IDX_STAGE)]],
            rows.at[s], sem).wait()

    def start_scatter(j, s):
        pltpu.async_copy(
            rows.at[s],
            acc.at[ridx.at[lax.rem(j // IDX_STAGE, 2), lax.rem(j, IDX_STAGE)]],
            sem_s, add=True)

    def wait_scatter(j, s):
        pltpu.make_async_copy(
            rows.at[s],
            acc.at[ridx.at[lax.rem(j // IDX_STAGE, 2), lax.rem(j, IDX_STAGE)]],
            sem_s).wait()

    stage_start(0)

    for i in range(ZROWS):
        for j in range(D // 16):
            zbuf[i, pl.ds(j * 16, 16)] = jnp.zeros((16,), jnp.float32)

    @pl.when((c == 0) & (w < NS - 1))
    def _():
        pltpu.sync_copy(x_hbm.at[pl.ds(w * ROWS_PER_TILE, ROWS_PER_TILE)],
                        acc.at[pl.ds(w * ROWS_PER_TILE, ROWS_PER_TILE)])

    @pl.when((c == 0) & (w == NS - 1))
    def _():
        pltpu.sync_copy(x_hbm.at[pl.ds(w * ROWS_PER_TILE, X_TAIL)],
                        acc.at[pl.ds(w * ROWS_PER_TILE, X_TAIL)])
        for k in range((ROWS_PER_TILE - X_TAIL) // ZROWS):
            pltpu.sync_copy(
                zbuf, acc.at[pl.ds(w * ROWS_PER_TILE + X_TAIL + k * ZROWS, ZROWS)])

    @pl.when(c != 0)
    def _():
        for k in range(ROWS_PER_TILE // ZROWS):
            pltpu.sync_copy(
                zbuf, acc.at[pl.ds(w * ROWS_PER_TILE + k * ZROWS, ZROWS)])

    stage_wait(0)
    for p in range(NBUF - 1):
        start_gather(p, p)

    plsc.subcore_barrier()

    def step(j, carry):
        s = lax.rem(j, NBUF)
        wait_gather(j, s)
        start_scatter(j, s)

        @pl.when(j >= 1)
        def _():
            wait_scatter(j - 1, lax.rem(j - 1, NBUF))

        @pl.when((lax.rem(j, IDX_STAGE) == NBUF) & (j + IDX_STAGE - NBUF < CPT))
        def _():
            stage_start(j // IDX_STAGE + 1)

        jn = j + NBUF - 1

        @pl.when(jn < CPT)
        def _():
            @pl.when(lax.rem(jn, IDX_STAGE) == 0)
            def _():
                stage_wait(jn // IDX_STAGE)

            start_gather(jn, lax.rem(jn, NBUF))
        return carry

    lax.fori_loop(0, CPT, step, 0)
    wait_scatter(CPT - 1, (CPT - 1) % NBUF)

    plsc.subcore_barrier()

    pltpu.sync_copy(acc.at[pl.ds(w * ROWS_PER_TILE, ROWS_PER_TILE)],
                    out_hbm.at[c, pl.ds(w * ROWS_PER_TILE, ROWS_PER_TILE)])


@functools.partial(
    pl.kernel,
    out_type=jax.ShapeDtypeStruct((NC, ACC_ROWS, D), jnp.float32),
    mesh=plsc.VectorSubcoreMesh(core_axis_name="c", subcore_axis_name="s"),
    scratch_types=[
        pltpu.VMEM_SHARED((ACC_ROWS, D), jnp.float32),
        pltpu.VMEM((2, IDX_STAGE, CHUNK), jnp.int32),
        pltpu.VMEM((2, IDX_STAGE, CHUNK), jnp.int32),
        pltpu.VMEM((NBUF, CHUNK, D), jnp.float32),
        pltpu.VMEM((ZROWS, D), jnp.float32),
        pltpu.SemaphoreType.DMA,
        pltpu.SemaphoreType.DMA,
        pltpu.SemaphoreType.DMA,
    ],
)
def _mpnn_sc(s2, r2, x_hbm, out_hbm, acc, sidx, ridx, rows, zbuf, sem, sem_s,
             sem_i):
    _sc_body(s2, r2, x_hbm, out_hbm, acc, sidx, ridx, rows, zbuf, sem, sem_s,
             sem_i)


def _combine_body(p_ref, o_ref):
    o_ref[...] = p_ref[0] + p_ref[1]


_combine = pl.pallas_call(
    _combine_body,
    grid=(10,),
    in_specs=[pl.BlockSpec((NC, N_NODES // 10, D), lambda i: (0, i, 0))],
    out_specs=pl.BlockSpec((N_NODES // 10, D), lambda i: (i, 0)),
    out_shape=jax.ShapeDtypeStruct((N_NODES, D), jnp.float32),
)


def kernel(V, E, X):
    pad = E_PAD - N_EDGES
    pad_send = jnp.arange(pad, dtype=jnp.int32) % N_NODES
    pad_recv = N_NODES + jnp.arange(pad, dtype=jnp.int32) % (ACC_ROWS - N_NODES)
    senders = jnp.concatenate([E[0], pad_send]).reshape(N_CHUNKS, CHUNK)
    receivers = jnp.concatenate([E[1], pad_recv]).reshape(N_CHUNKS, CHUNK)
    partials = _mpnn_sc(senders, receivers, X)
    return _combine(partials)
